# baseline (speedup 1.0000x reference)
import functools

import jax
import jax.numpy as jnp
from jax import lax
from jax.experimental import pallas as pl
from jax.experimental.pallas import tpu as pltpu
from jax.experimental.pallas import tpu_sc as plsc

_GRID = 1000
_NTILES = _GRID * _GRID
_D = 64
_DW = _D // 2
_B, _L = 4096, 200
_TOTAL = _B * _L
_NC, _NS = 2, 16
_NW = _NC * _NS
_RPW = _TOTAL // _NW
_C = 1280
_NCHUNK = _RPW // _C
_GPC = _C // 16
_NGATHER = _C // 128


@functools.partial(
    pl.kernel,
    mesh=plsc.VectorSubcoreMesh(core_axis_name="c", subcore_axis_name="s"),
    compiler_params=pltpu.CompilerParams(
        needs_layout_passes=False, use_tc_tiling_on_sc=False
    ),
    out_type=jax.ShapeDtypeStruct((_TOTAL, _DW), jnp.float32),
    scratch_types=[
        pltpu.VMEM((2 * _C,), jnp.float32),
        pltpu.VMEM((_C,), jnp.int32),
        pltpu.VMEM((_C, _DW), jnp.float32),
        pltpu.SemaphoreType.DMA,
    ],
)
def _sc_lookup(coords_hbm, table_hbm, out_hbm, cbuf, ibuf, rbuf, sem):
    wid = lax.axis_index("s") * _NC + lax.axis_index("c")
    base_row = wid * _RPW

    def chunk_body(i, carry):
        row0 = base_row + i * _C
        pltpu.sync_copy(coords_hbm.at[pl.ds(2 * row0, 2 * _C)], cbuf)

        def group_body(g, c):
            lane = lax.iota(jnp.int32, 16)
            xi = lane * 2 + g * 32
            xs = plsc.load_gather(cbuf, [xi])
            ys = plsc.load_gather(cbuf, [xi + 1])
            tx = (xs * (1.0 / 256.0)).astype(jnp.int32)
            ty = (ys * (1.0 / 256.0)).astype(jnp.int32)
            idx = tx * _GRID + ty
            idx = jnp.minimum(jnp.maximum(idx, 0), _NTILES - 1)
            ibuf[pl.ds(g * 16, 16)] = idx
            return c

        lax.fori_loop(0, _GPC, group_body, 0, unroll=2)

        copies = [
            pltpu.async_copy(
                table_hbm.at[ibuf.at[pl.ds(k * 128, 128)]],
                rbuf.at[pl.ds(k * 128, 128)],
                sem,
            )
            for k in range(_NGATHER)
        ]
        for cp in copies:
            cp.wait()
        pltpu.sync_copy(rbuf, out_hbm.at[pl.ds(row0, _C)])
        return carry

    lax.fori_loop(0, _NCHUNK, chunk_body, 0)


def kernel(coords, pos_embed):
    coords_flat = coords.reshape(_TOTAL * 2)
    table = lax.bitcast_convert_type(
        pos_embed[0].reshape(_NTILES, _DW, 2), jnp.float32
    )
    out32 = _sc_lookup(coords_flat, table)
    out = lax.bitcast_convert_type(out32, jnp.float16)
    return out.reshape(_B, _L, _D)


# speedup vs baseline: 2.6419x; 2.6419x over previous
import functools

import jax
import jax.numpy as jnp
from jax import lax
from jax.experimental import pallas as pl
from jax.experimental.pallas import tpu as pltpu
from jax.experimental.pallas import tpu_sc as plsc

_GRID = 1000
_NTILES = _GRID * _GRID
_D = 64
_B, _L = 4096, 200
_NBT = _B // 128
_NC, _NS = 2, 16
_NW = _NC * _NS
_WAVE = 8
_NWAVE = _NBT // _WAVE


@functools.partial(
    pl.kernel,
    mesh=plsc.VectorSubcoreMesh(core_axis_name="c", subcore_axis_name="s"),
    compiler_params=pltpu.CompilerParams(
        needs_layout_passes=False, use_tc_tiling_on_sc=False
    ),
    out_type=jax.ShapeDtypeStruct((_B, _L * _D), jnp.float16),
    scratch_types=[
        pltpu.VMEM((_NBT, 2, 128), jnp.float32),
        pltpu.VMEM((_B,), jnp.int32),
        pltpu.VMEM((2, _WAVE, 128, _D), jnp.float16),
        pltpu.SemaphoreType.DMA,
        pltpu.SemaphoreType.DMA,
    ],
)
def _sc_lookup(coords_hbm, table_hbm, out_hbm, cbuf, ibuf, rbuf, gsem, osem):
    wid = lax.axis_index("s") * _NC + lax.axis_index("c")
    n_l = jnp.where(wid < _L - 6 * _NW, 7, 6)

    def l_body(kk, carry):
        l = wid + kk * _NW
        pltpu.sync_copy(coords_hbm.at[l], cbuf)

        def group_body(g, c):
            tb = g // 8
            gg = g % 8
            xs = cbuf[tb, 0, pl.ds(gg * 16, 16)]
            ys = cbuf[tb, 1, pl.ds(gg * 16, 16)]
            tx = (xs * (1.0 / 256.0)).astype(jnp.int32)
            ty = (ys * (1.0 / 256.0)).astype(jnp.int32)
            idx = tx * _GRID + ty
            idx = jnp.minimum(jnp.maximum(idx, 0), _NTILES - 1)
            ibuf[pl.ds(g * 16, 16)] = idx
            return c

        lax.fori_loop(0, _B // 16, group_body, 0, unroll=2)

        out_copies = []
        for wave in range(_NWAVE):
            p = wave % 2
            if wave >= 2:
                for _ in range(_WAVE):
                    out_copies.pop(0).wait()
            gathers = []
            for k in range(_WAVE):
                bt = wave * _WAVE + k
                gathers.append(
                    pltpu.async_copy(
                        table_hbm.at[ibuf.at[pl.ds(bt * 128, 128)]],
                        rbuf.at[p, k],
                        gsem,
                    )
                )
            for cp in gathers:
                cp.wait()
            for k in range(_WAVE):
                bt = wave * _WAVE + k
                out_copies.append(
                    pltpu.async_copy(
                        rbuf.at[p, k],
                        out_hbm.at[pl.ds(bt * 128, 128), pl.ds(l * _D, _D)],
                        osem,
                    )
                )
        for cp in out_copies:
            cp.wait()
        return carry

    lax.fori_loop(0, n_l, l_body, 0)


def kernel(coords, pos_embed):
    x = coords.reshape(_NBT, 128, _L, 2).transpose(2, 0, 3, 1)
    table = pos_embed[0]
    out2 = _sc_lookup(x, table)
    return out2.reshape(_B, _L, _D)
